# initial kernel scaffold (unmeasured)
import jax
import jax.numpy as jnp
from jax import lax
from jax.experimental import pallas as pl
from jax.experimental.pallas import tpu as pltpu

N_DEV = 32


def kernel(x, w_mat):
    m_per, k = x.shape
    _, n = w_mat.shape
    n_per = n // N_DEV

    def body(x_ref, w_ref, out_ref,
             y_blocks, data_recv, amax_send, amax_recv,
             data_send_sems, data_recv_sems,
             amax_send_sems, amax_recv_sems):
        my = lax.axis_index("i")

        xb = x_ref[...].astype(jnp.bfloat16)
        kc = 512
        acc = jnp.zeros((m_per, n), jnp.float32)
        for kk in range(0, k, kc):
            acc = acc + jnp.dot(
                xb[:, kk:kk + kc],
                w_ref[kk:kk + kc, :].astype(jnp.bfloat16),
                preferred_element_type=jnp.float32,
            )
        y = jnp.maximum(acc, 0.0)
        local_amax = jnp.max(y)

        for j in range(N_DEV):
            y_blocks[j, :, :] = y[:, j * n_per:(j + 1) * n_per].astype(
                jnp.bfloat16)
        amax_send[...] = jnp.full((8, 128), local_amax, jnp.float32)

        own = lax.dynamic_slice(y, (0, my * n_per), (m_per, n_per))
        data_recv[pl.ds(my, 1)] = own.astype(jnp.bfloat16)[None]
        amax_recv[pl.ds(my, 1)] = jnp.full((1, 8, 128), local_amax,
                                           jnp.float32)

        barrier = pltpu.get_barrier_semaphore()
        for dp in range(1, N_DEV):
            tgt = lax.rem(my + dp, N_DEV)
            pl.semaphore_signal(barrier, inc=1, device_id=(tgt,),
                                device_id_type=pl.DeviceIdType.MESH)
        pl.semaphore_wait(barrier, N_DEV - 1)

        descs = []
        for dp in range(1, N_DEV):
            tgt = lax.rem(my + dp, N_DEV)
            d = pltpu.make_async_remote_copy(
                src_ref=y_blocks.at[tgt],
                dst_ref=data_recv.at[my],
                send_sem=data_send_sems.at[dp],
                recv_sem=data_recv_sems.at[my],
                device_id=(tgt,),
                device_id_type=pl.DeviceIdType.MESH,
            )
            d.start()
            a = pltpu.make_async_remote_copy(
                src_ref=amax_send,
                dst_ref=amax_recv.at[my],
                send_sem=amax_send_sems.at[dp],
                recv_sem=amax_recv_sems.at[my],
                device_id=(tgt,),
                device_id_type=pl.DeviceIdType.MESH,
            )
            a.start()
            descs.append((d, a))

        for p in range(N_DEV):
            @pl.when(p != my)
            def _(p=p):
                wr = pltpu.make_async_remote_copy(
                    src_ref=y_blocks.at[p],
                    dst_ref=data_recv.at[p],
                    send_sem=data_send_sems.at[0],
                    recv_sem=data_recv_sems.at[p],
                    device_id=(p,),
                    device_id_type=pl.DeviceIdType.MESH,
                )
                wr.wait_recv()
                wa = pltpu.make_async_remote_copy(
                    src_ref=amax_send,
                    dst_ref=amax_recv.at[p],
                    send_sem=amax_send_sems.at[0],
                    recv_sem=amax_recv_sems.at[p],
                    device_id=(p,),
                    device_id_type=pl.DeviceIdType.MESH,
                )
                wa.wait_recv()

        for d, a in descs:
            d.wait_send()
            a.wait_send()

        gmax = jnp.max(amax_recv[...])
        scale = gmax / 448.0
        blocks = data_recv[...].astype(jnp.float32)
        q = jnp.minimum(blocks / scale, 448.0)
        q = q.astype(jnp.float8_e4m3fn).astype(jnp.float32) * scale
        out_ref[...] = q.reshape(N_DEV * m_per, n_per)

    return pl.pallas_call(
        body,
        out_shape=jax.ShapeDtypeStruct((N_DEV * m_per, n_per), jnp.float32),
        in_specs=[
            pl.BlockSpec(memory_space=pltpu.VMEM),
            pl.BlockSpec(memory_space=pltpu.VMEM),
        ],
        out_specs=pl.BlockSpec(memory_space=pltpu.VMEM),
        scratch_shapes=[
            pltpu.VMEM((N_DEV, m_per, n_per), jnp.bfloat16),
            pltpu.VMEM((N_DEV, m_per, n_per), jnp.bfloat16),
            pltpu.VMEM((8, 128), jnp.float32),
            pltpu.VMEM((N_DEV, 8, 128), jnp.float32),
            pltpu.SemaphoreType.DMA((N_DEV,)),
            pltpu.SemaphoreType.DMA((N_DEV,)),
            pltpu.SemaphoreType.DMA((N_DEV,)),
            pltpu.SemaphoreType.DMA((N_DEV,)),
        ],
        compiler_params=pltpu.CompilerParams(
            collective_id=0,
            vmem_limit_bytes=128 * 1024 * 1024,
        ),
    )(x, w_mat)


# baseline (device time: 42390 ns/iter reference)
import jax
import jax.numpy as jnp
from jax import lax
from jax.experimental import pallas as pl
from jax.experimental.pallas import tpu as pltpu

N_DEV = 32


def kernel(x, w_mat):
    m_per, k = x.shape
    _, n = w_mat.shape
    n_per = n // N_DEV

    def body(x_ref, w_ref, out_ref,
             y_blocks, data_recv, amax_send, amax_recv,
             data_send_sems, data_recv_sems,
             amax_send_sems, amax_recv_sems):
        my = lax.axis_index("i")

        xb = x_ref[...].astype(jnp.bfloat16)
        kc = 512
        acc = jnp.zeros((m_per, n), jnp.float32)
        for kk in range(0, k, kc):
            acc = acc + jnp.dot(
                xb[:, kk:kk + kc],
                w_ref[kk:kk + kc, :].astype(jnp.bfloat16),
                preferred_element_type=jnp.float32,
            )
        y = jnp.maximum(acc, 0.0)
        local_amax = jnp.max(y)

        for j in range(N_DEV):
            y_blocks[j, :, :] = y[:, j * n_per:(j + 1) * n_per].astype(
                jnp.bfloat16)
        amax_send[...] = jnp.full((8, 128), local_amax, jnp.float32)

        data_recv[pl.ds(my, 1)] = y_blocks[pl.ds(my, 1)]
        amax_recv[pl.ds(my, 1)] = jnp.full((1, 8, 128), local_amax,
                                           jnp.float32)

        barrier = pltpu.get_barrier_semaphore()
        for dp in range(1, N_DEV):
            tgt = lax.rem(my + dp, N_DEV)
            pl.semaphore_signal(barrier, inc=1, device_id=(tgt,),
                                device_id_type=pl.DeviceIdType.MESH)
        pl.semaphore_wait(barrier, N_DEV - 1)

        descs = []
        for dp in range(1, N_DEV):
            tgt = lax.rem(my + dp, N_DEV)
            d = pltpu.make_async_remote_copy(
                src_ref=y_blocks.at[tgt],
                dst_ref=data_recv.at[my],
                send_sem=data_send_sems.at[dp],
                recv_sem=data_recv_sems.at[my],
                device_id=(tgt,),
                device_id_type=pl.DeviceIdType.MESH,
            )
            d.start()
            a = pltpu.make_async_remote_copy(
                src_ref=amax_send,
                dst_ref=amax_recv.at[my],
                send_sem=amax_send_sems.at[dp],
                recv_sem=amax_recv_sems.at[my],
                device_id=(tgt,),
                device_id_type=pl.DeviceIdType.MESH,
            )
            a.start()
            descs.append((d, a))

        for p in range(N_DEV):
            @pl.when(p != my)
            def _(p=p):
                wr = pltpu.make_async_remote_copy(
                    src_ref=y_blocks.at[p],
                    dst_ref=data_recv.at[p],
                    send_sem=data_send_sems.at[0],
                    recv_sem=data_recv_sems.at[p],
                    device_id=(p,),
                    device_id_type=pl.DeviceIdType.MESH,
                )
                wr.wait_recv()
                wa = pltpu.make_async_remote_copy(
                    src_ref=amax_send,
                    dst_ref=amax_recv.at[p],
                    send_sem=amax_send_sems.at[0],
                    recv_sem=amax_recv_sems.at[p],
                    device_id=(p,),
                    device_id_type=pl.DeviceIdType.MESH,
                )
                wa.wait_recv()

        for d, a in descs:
            d.wait_send()
            a.wait_send()

        gmax = jnp.max(amax_recv[...])
        scale = gmax / 448.0
        blocks = data_recv[...].astype(jnp.float32)
        q = jnp.minimum(blocks / scale, 448.0)
        q = q.astype(jnp.float8_e4m3fn).astype(jnp.float32) * scale
        out_ref[...] = q.reshape(N_DEV * m_per, n_per)

    return pl.pallas_call(
        body,
        out_shape=jax.ShapeDtypeStruct((N_DEV * m_per, n_per), jnp.float32),
        in_specs=[
            pl.BlockSpec(memory_space=pltpu.VMEM),
            pl.BlockSpec(memory_space=pltpu.VMEM),
        ],
        out_specs=pl.BlockSpec(memory_space=pltpu.VMEM),
        scratch_shapes=[
            pltpu.VMEM((N_DEV, m_per, n_per), jnp.bfloat16),
            pltpu.VMEM((N_DEV, m_per, n_per), jnp.bfloat16),
            pltpu.VMEM((8, 128), jnp.float32),
            pltpu.VMEM((N_DEV, 8, 128), jnp.float32),
            pltpu.SemaphoreType.DMA((N_DEV,)),
            pltpu.SemaphoreType.DMA((N_DEV,)),
            pltpu.SemaphoreType.DMA((N_DEV,)),
            pltpu.SemaphoreType.DMA((N_DEV,)),
        ],
        compiler_params=pltpu.CompilerParams(
            collective_id=0,
            vmem_limit_bytes=128 * 1024 * 1024,
        ),
    )(x, w_mat)
